# baseline (speedup 1.0000x reference)
import functools

import jax
import jax.numpy as jnp
import numpy as np
from jax import lax
from jax.experimental import pallas as pl
from jax.experimental.pallas import tpu as pltpu
from jax.experimental.pallas import tpu_sc as plsc

B = 4096
S = 200
E = 64
H = 512
C = 128

NW = 32
NC = 2
NSUB = 16
BPW = B // NW
CH = 128
NCHUNK = BPW * S // CH
NB = 4

SLOTS = NSUB * BPW


def _scatter_slots():
    sub = np.arange(NW) // NC
    flat = np.arange(BPW * S) // S
    slots = sub[:, None] * BPW + flat[None, :]
    return slots.reshape(NW, NCHUNK, CH).astype(np.int32)


_SLOTS_CONST = _scatter_slots()


def _pool_sc(x_r, table):
    mesh = plsc.VectorSubcoreMesh(core_axis_name="c", subcore_axis_name="s")

    @functools.partial(
        pl.kernel,
        out_type=jax.ShapeDtypeStruct((B, E), jnp.float32),
        mesh=mesh,
        compiler_params=pltpu.CompilerParams(use_tc_tiling_on_sc=False),
        scratch_types=[
            pltpu.VMEM((NCHUNK, CH), jnp.int32),
            pltpu.VMEM((NCHUNK, CH), jnp.int32),
            pltpu.VMEM((NB, CH, E), jnp.float32),
            pltpu.VMEM_SHARED((SLOTS, E), jnp.float32),
            pltpu.SemaphoreType.DMA,
            pltpu.SemaphoreType.DMA,
            pltpu.SemaphoreType.DMA,
            pltpu.SemaphoreType.DMA,
            pltpu.SemaphoreType.DMA,
            pltpu.SemaphoreType.DMA,
            pltpu.SemaphoreType.DMA,
            pltpu.SemaphoreType.DMA,
        ],
    )
    def k(x_hbm, slots_hbm, table_hbm, out_hbm, idx_v, sl_v, bufs, acc_sh,
          g0, g1, g2, g3, s0, s1, s2, s3):
        wid = lax.axis_index("s") * NC + lax.axis_index("c")
        sid = lax.axis_index("s")
        gsem = [g0, g1, g2, g3]
        ssem = [s0, s1, s2, s3]

        pltpu.sync_copy(x_hbm.at[wid], idx_v)
        pltpu.sync_copy(slots_hbm.at[wid], sl_v)

        def zbody(r, carry):
            z = jnp.zeros((16,), jnp.float32)
            for cc in range(4):
                bufs[0, r, pl.ds(16 * cc, 16)] = z
            return carry
        lax.fori_loop(0, CH, zbody, 0)
        pltpu.sync_copy(bufs.at[0], acc_sh.at[pl.ds(sid * BPW, BPW)])

        def gstart(j, b):
            pltpu.make_async_copy(
                table_hbm.at[idx_v.at[j]], bufs.at[b], gsem[b]).start()

        def gwait(j, b):
            pltpu.make_async_copy(
                table_hbm.at[idx_v.at[j]], bufs.at[b], gsem[b]).wait()

        def sstart(j, b):
            pltpu.make_async_copy(
                bufs.at[b], acc_sh.at[sl_v.at[j]], ssem[b]).start(add=True)

        def swait(j, b):
            pltpu.make_async_copy(
                bufs.at[b], acc_sh.at[sl_v.at[j]], ssem[b]).wait()

        gstart(0, 0)
        gstart(1, 1)

        def outer(g, carry):
            for bb in range(NB):
                j = NB * g + bb
                nb = (bb + 2) % NB
                gwait(j, bb)
                sstart(j, bb)

                @pl.when(jnp.logical_and(j >= 2, j + 2 < NCHUNK))
                def _():
                    swait(j - 2, nb)

                @pl.when(j + 2 < NCHUNK)
                def _():
                    gstart(j + 2, nb)
            return carry

        lax.fori_loop(0, NCHUNK // NB, outer, 0)
        for bb in range(NB):
            swait(NCHUNK - NB + bb, bb)

        pltpu.sync_copy(acc_sh.at[pl.ds(sid * BPW, BPW)],
                        out_hbm.at[pl.ds(wid * BPW, BPW)])

    return k(x_r, jnp.asarray(_SLOTS_CONST), table)


def _mlp_tc(pooled, W1, b1, W2, b2):
    BLK = 512

    def body(p_ref, w1_ref, b1_ref, w2_ref, b2_ref, o_ref):
        p = p_ref[...] * (1.0 / S)
        h = jnp.dot(p, w1_ref[...], preferred_element_type=jnp.float32)
        h = jnp.maximum(h + b1_ref[...], 0.0)
        o_ref[...] = jnp.dot(h, w2_ref[...],
                             preferred_element_type=jnp.float32) + b2_ref[...]

    return pl.pallas_call(
        body,
        grid=(B // BLK,),
        in_specs=[
            pl.BlockSpec((BLK, E), lambda i: (i, 0)),
            pl.BlockSpec((E, H), lambda i: (0, 0)),
            pl.BlockSpec((1, H), lambda i: (0, 0)),
            pl.BlockSpec((H, C), lambda i: (0, 0)),
            pl.BlockSpec((1, C), lambda i: (0, 0)),
        ],
        out_specs=pl.BlockSpec((BLK, C), lambda i: (i, 0)),
        out_shape=jax.ShapeDtypeStruct((B, C), jnp.float32),
    )(pooled, W1, b1.reshape(1, H), W2, b2.reshape(1, C))


def kernel(x, table, W1, b1, W2, b2):
    x_r = x.astype(jnp.int32).reshape(NW, NCHUNK, CH)
    pooled = _pool_sc(x_r, table)
    return _mlp_tc(pooled, W1, b1, W2, b2)


# speedup vs baseline: 1.1000x; 1.1000x over previous
import functools

import jax
import jax.numpy as jnp
from jax import lax
from jax.experimental import pallas as pl
from jax.experimental.pallas import tpu as pltpu
from jax.experimental.pallas import tpu_sc as plsc

B = 4096
S = 200
E = 64
H = 512
C = 128

NW = 32
NC = 2
BPW = B // NW
IPW = BPW * S
CH0 = 128
CH1 = S - CH0


def _pool_sc(x_flat, table):
    mesh = plsc.VectorSubcoreMesh(core_axis_name="c", subcore_axis_name="s")

    @functools.partial(
        pl.kernel,
        out_type=jax.ShapeDtypeStruct((B, E), jnp.float32),
        mesh=mesh,
        compiler_params=pltpu.CompilerParams(use_tc_tiling_on_sc=False),
        scratch_types=[
            pltpu.VMEM((IPW,), jnp.int32),
            pltpu.VMEM((2, S, E), jnp.float32),
            pltpu.VMEM((BPW, E), jnp.float32),
            pltpu.SemaphoreType.DMA,
            pltpu.SemaphoreType.DMA,
        ],
    )
    def k(x_hbm, table_hbm, out_hbm, idx_v, rows_v, acc_v, sem0, sem1):
        wid = lax.axis_index("s") * NC + lax.axis_index("c")
        pltpu.sync_copy(x_hbm.at[pl.ds(wid * IPW, IPW)], idx_v)
        sems = [sem0, sem1]

        def issue(i, b):
            base = i * S
            pltpu.make_async_copy(
                table_hbm.at[idx_v.at[pl.ds(base, CH0)]],
                rows_v.at[b, pl.ds(0, CH0)], sems[b]).start()
            pltpu.make_async_copy(
                table_hbm.at[idx_v.at[pl.ds(base + CH0, CH1)]],
                rows_v.at[b, pl.ds(CH0, CH1)], sems[b]).start()

        def wait_g(i, b):
            base = i * S
            pltpu.make_async_copy(
                table_hbm.at[idx_v.at[pl.ds(base, CH0)]],
                rows_v.at[b, pl.ds(0, CH0)], sems[b]).wait()
            pltpu.make_async_copy(
                table_hbm.at[idx_v.at[pl.ds(base + CH0, CH1)]],
                rows_v.at[b, pl.ds(CH0, CH1)], sems[b]).wait()

        def reduce_into(i, b):
            def body(r, carry):
                out = []
                for c in range(4):
                    v0 = rows_v[b, 2 * r, pl.ds(16 * c, 16)]
                    v1 = rows_v[b, 2 * r + 1, pl.ds(16 * c, 16)]
                    out.append(carry[c] + v0 + v1)
                return tuple(out)
            init = tuple(jnp.zeros((16,), jnp.float32) for _ in range(4))
            acc = lax.fori_loop(0, S // 2, body, init)
            for c in range(4):
                acc_v[i, pl.ds(16 * c, 16)] = acc[c]

        issue(0, 0)
        issue(1, 1)

        def outer(g, carry):
            for bb in range(2):
                i = 2 * g + bb
                wait_g(i, bb)

                @pl.when(i + 2 < BPW)
                def _():
                    issue(i + 2, bb)

                reduce_into(i, bb)
            return carry

        lax.fori_loop(0, BPW // 2, outer, 0)
        pltpu.sync_copy(acc_v, out_hbm.at[pl.ds(wid * BPW, BPW)])

    return k(x_flat, table)


def _mlp_tc(pooled, W1, b1, W2, b2):
    BLK = 512

    def body(p_ref, w1_ref, b1_ref, w2_ref, b2_ref, o_ref):
        p = p_ref[...] * (1.0 / S)
        h = jnp.dot(p, w1_ref[...], preferred_element_type=jnp.float32)
        h = jnp.maximum(h + b1_ref[...], 0.0)
        o_ref[...] = jnp.dot(h, w2_ref[...],
                             preferred_element_type=jnp.float32) + b2_ref[...]

    return pl.pallas_call(
        body,
        grid=(B // BLK,),
        in_specs=[
            pl.BlockSpec((BLK, E), lambda i: (i, 0)),
            pl.BlockSpec((E, H), lambda i: (0, 0)),
            pl.BlockSpec((1, H), lambda i: (0, 0)),
            pl.BlockSpec((H, C), lambda i: (0, 0)),
            pl.BlockSpec((1, C), lambda i: (0, 0)),
        ],
        out_specs=pl.BlockSpec((BLK, C), lambda i: (i, 0)),
        out_shape=jax.ShapeDtypeStruct((B, C), jnp.float32),
    )(pooled, W1, b1.reshape(1, H), W2, b2.reshape(1, C))


def kernel(x, table, W1, b1, W2, b2):
    x_flat = x.astype(jnp.int32).reshape(B * S)
    pooled = _pool_sc(x_flat, table)
    return _mlp_tc(pooled, W1, b1, W2, b2)
